# baseline (speedup 1.0000x reference)
import jax
import jax.numpy as jnp
from jax import lax
from jax.experimental import pallas as pl
from jax.experimental.pallas import tpu as pltpu
from jax.experimental.pallas import tpu_sc as plsc

B = 16384
CTX = 20
EMB = 64
VOCAB = 1000000
TOTAL = B * CTX
NC = 2
NS = 16
NW = NC * NS

CPC = 4
CHUNK_V = CPC * 128
FULL_COLS = VOCAB // 128
NCHUNKS = FULL_COLS // CPC
ROWS_PC = CHUNK_V // 2
TAIL_V = VOCAB - FULL_COLS * 128
FMT_ROWS = VOCAB // 2

BLOCKS = B // 8
BPW = BLOCKS // NW
BLK_IDX = 8 * CTX
PER_W = BPW * BLK_IDX


def _fmt_compute(inb, pack, nrows):
    iota = lax.iota(jnp.int32, 16)
    par64 = jnp.left_shift(jnp.bitwise_and(iota, 1), 6)

    @plsc.parallel_loop(0, (2 * nrows) // 16, unroll=2)
    def mgroup(m):
        vbase = m * 16
        prow = lax.shift_right_logical(vbase + iota, 1)
        for e in range(EMB):
            vals = inb[e, pl.ds(vbase, 16)]
            plsc.store_scatter(pack, [prow, par64 + e], vals)


def _fmt_body(tT_hbm, tail_hbm, fmt_hbm, inb0, inb1, pack, sem0, sem1):
    wid = lax.axis_index("s") * NC + lax.axis_index("c")
    start = jnp.where(wid > 0, wid * 61 + 1, 0)
    count = jnp.where(wid > 0, 61, 62)
    inb = (inb0, inb1)
    sems = (sem0, sem1)

    def start_in(ci, b):
        c0 = (start + ci) * CHUNK_V
        pltpu.async_copy(tT_hbm.at[:, pl.ds(c0, CHUNK_V)], inb[b], sems[b])

    start_in(0, 0)

    def group(gp, carry):
        for b in range(2):
            ci = gp * 2 + b

            @pl.when(ci < count)
            def _():
                @pl.when(ci + 1 < count)
                def _():
                    start_in(ci + 1, 1 - b)
                pltpu.make_async_copy(
                    tT_hbm.at[:, pl.ds(0, CHUNK_V)], inb[b], sems[b]).wait()
                _fmt_compute(inb[b], pack, ROWS_PC)
                r0 = (start + ci) * ROWS_PC
                pltpu.sync_copy(pack, fmt_hbm.at[pl.ds(r0, ROWS_PC)])
        return carry

    lax.fori_loop(0, 31, group, 0)

    @pl.when(wid == NW - 1)
    def _():
        pltpu.sync_copy(tail_hbm, inb0.at[:, pl.ds(0, 128)])
        _fmt_compute(inb0, pack, TAIL_V // 2)
        pltpu.sync_copy(pack.at[pl.ds(0, TAIL_V // 2)],
                        fmt_hbm.at[pl.ds(FULL_COLS * 64, TAIL_V // 2)])


def _gather_body(x_hbm, fmt_hbm, out_hbm, idx_v, pidx0, pidx1,
                 rows0, rows1, pack0, pack1, semg0, semg1):
    wid = lax.axis_index("s") * NC + lax.axis_index("c")
    base = wid * PER_W
    pltpu.sync_copy(x_hbm.at[pl.ds(pl.multiple_of(base, PER_W), PER_W)],
                    idx_v)
    pidx = (pidx0, pidx1)
    rows = (rows0, rows1)
    packs = (pack0, pack1)
    semg = (semg0, semg1)
    iota = lax.iota(jnp.int32, 16)

    def calc_pidx(k, b):
        def vec(m, carry):
            j = idx_v[pl.ds(k * BLK_IDX + m * 16, 16)]
            pidx[b][pl.ds(m * 16, 16)] = lax.shift_right_logical(j, 1)
            return carry
        lax.fori_loop(0, BLK_IDX // 16, vec, 0)

    def start_gather(b):
        pltpu.async_copy(fmt_hbm.at[pidx[b]], rows[b], semg[b])

    def pack_block(k, b):
        @plsc.parallel_loop(0, BLK_IDX, unroll=8)
        def row(g):
            jvec = plsc.load_gather(
                idx_v, [jnp.full((16,), k * BLK_IDX + g, jnp.int32)])
            off = jnp.left_shift(jnp.bitwise_and(jvec, 1), 6)
            grow = jnp.full((16,), g, jnp.int32)
            prow = jnp.full((16,), g // CTX, jnp.int32)
            cbase = (g % CTX) * EMB
            for q in range(4):
                vals = plsc.load_gather(rows[b], [grow, off + (iota + q * 16)])
                plsc.store_scatter(packs[b],
                                   [prow, iota + (cbase + q * 16)], vals)

    calc_pidx(0, 0)
    start_gather(0)

    def group(gp, carry):
        for b in range(2):
            k = gp * 2 + b

            @pl.when(k + 1 < BPW)
            def _():
                calc_pidx(k + 1, 1 - b)
                start_gather(1 - b)
            pltpu.make_async_copy(fmt_hbm.at[pidx[b]], rows[b],
                                  semg[b]).wait()
            pack_block(k, b)
            row0 = pl.multiple_of((wid * BPW + k) * 8, 8)
            pltpu.sync_copy(packs[b], out_hbm.at[pl.ds(row0, 8)])
        return carry

    lax.fori_loop(0, BPW // 2, group, 0)


def kernel(x, table):
    xp = x.astype(jnp.int32).reshape(-1)
    mesh = plsc.VectorSubcoreMesh(core_axis_name="c", subcore_axis_name="s")

    fmt = pl.kernel(
        _fmt_body,
        mesh=mesh,
        out_type=jax.ShapeDtypeStruct((FMT_ROWS, 128), jnp.float32),
        scratch_types=[
            pltpu.VMEM((EMB, CHUNK_V), jnp.float32),
            pltpu.VMEM((EMB, CHUNK_V), jnp.float32),
            pltpu.VMEM((ROWS_PC, 128), jnp.float32),
            pltpu.SemaphoreType.DMA,
            pltpu.SemaphoreType.DMA,
        ],
        compiler_params=pltpu.CompilerParams(use_tc_tiling_on_sc=True, needs_layout_passes=False),
    )(table.T, jnp.pad(table[FULL_COLS * 128:].T, ((0, 0), (0, 128 - TAIL_V))))

    out = pl.kernel(
        _gather_body,
        mesh=mesh,
        out_type=jax.ShapeDtypeStruct((B, CTX * EMB), jnp.float32),
        scratch_types=[
            pltpu.VMEM((PER_W,), jnp.int32),
            pltpu.VMEM((BLK_IDX,), jnp.int32),
            pltpu.VMEM((BLK_IDX,), jnp.int32),
            pltpu.VMEM((BLK_IDX, 128), jnp.float32),
            pltpu.VMEM((BLK_IDX, 128), jnp.float32),
            pltpu.VMEM((8, CTX * EMB), jnp.float32),
            pltpu.VMEM((8, CTX * EMB), jnp.float32),
            pltpu.SemaphoreType.DMA,
            pltpu.SemaphoreType.DMA,
        ],
        compiler_params=pltpu.CompilerParams(use_tc_tiling_on_sc=True, needs_layout_passes=False),
    )(xp, fmt)
    return out


# speedup vs baseline: 1.9569x; 1.9569x over previous
import jax
import jax.numpy as jnp
from jax import lax
from jax.experimental import pallas as pl
from jax.experimental.pallas import tpu as pltpu
from jax.experimental.pallas import tpu_sc as plsc

B = 16384
CTX = 20
EMB = 64
VOCAB = 1000000
TOTAL = B * CTX
NC = 2
NS = 16
NW = NC * NS

CPC = 4
CHUNK_V = CPC * 128
FULL_COLS = VOCAB // 128
NCHUNKS = FULL_COLS // CPC
ROWS_PC = CHUNK_V // 2
TAIL_V = VOCAB - FULL_COLS * 128
FMT_ROWS = VOCAB // 2

BLOCKS = B // 8
BPW = BLOCKS // NW
BLK_IDX = 8 * CTX
PER_W = BPW * BLK_IDX


def _fmt_compute(inb, pack, nrows):
    iota = lax.iota(jnp.int32, 16)
    par1 = jnp.bitwise_and(iota, 1)

    @plsc.parallel_loop(0, (2 * nrows) // 16, unroll=2)
    def mgroup(m):
        vbase = m * 16
        prow = lax.shift_right_logical(vbase + iota, 1)
        for e in range(EMB):
            vals = inb[e, pl.ds(vbase, 16)]
            plsc.store_scatter(pack, [prow, par1 + 2 * e], vals)


def _fmt_body(tT_hbm, tail_hbm, fmt_hbm, inb0, inb1, pack, sem0, sem1):
    wid = lax.axis_index("s") * NC + lax.axis_index("c")
    start = jnp.where(wid > 0, wid * 61 + 1, 0)
    count = jnp.where(wid > 0, 61, 62)
    inb = (inb0, inb1)
    sems = (sem0, sem1)

    def start_in(ci, b):
        c0 = (start + ci) * CHUNK_V
        pltpu.async_copy(tT_hbm.at[:, pl.ds(c0, CHUNK_V)], inb[b], sems[b])

    start_in(0, 0)

    def group(gp, carry):
        for b in range(2):
            ci = gp * 2 + b

            @pl.when(ci < count)
            def _():
                @pl.when(ci + 1 < count)
                def _():
                    start_in(ci + 1, 1 - b)
                pltpu.make_async_copy(
                    tT_hbm.at[:, pl.ds(0, CHUNK_V)], inb[b], sems[b]).wait()
                _fmt_compute(inb[b], pack, ROWS_PC)
                r0 = (start + ci) * ROWS_PC
                pltpu.sync_copy(pack.at[:, pl.ds(0, 128)],
                                fmt_hbm.at[pl.ds(r0, ROWS_PC)])
        return carry

    lax.fori_loop(0, 31, group, 0)

    @pl.when(wid == NW - 1)
    def _():
        pltpu.sync_copy(tail_hbm, inb0.at[:, pl.ds(0, 128)])
        _fmt_compute(inb0, pack, TAIL_V // 2)
        pltpu.sync_copy(pack.at[pl.ds(0, TAIL_V // 2), pl.ds(0, 128)],
                        fmt_hbm.at[pl.ds(FULL_COLS * 64, TAIL_V // 2)])


def _gather_body(x_hbm, fmt_hbm, out_hbm, idx_v, pidx0, pidx1,
                 rows0, rows1, pack0, pack1, semg0, semg1):
    wid = lax.axis_index("s") * NC + lax.axis_index("c")
    base = wid * PER_W
    pltpu.sync_copy(x_hbm.at[pl.ds(pl.multiple_of(base, PER_W), PER_W)],
                    idx_v)
    pidx = (pidx0, pidx1)
    rows = (rows0, rows1)
    packs = (pack0, pack1)
    semg = (semg0, semg1)
    iota = lax.iota(jnp.int32, 16)

    def calc_pidx(k, b):
        def vec(m, carry):
            j = idx_v[pl.ds(k * BLK_IDX + m * 16, 16)]
            pidx[b][pl.ds(m * 16, 16)] = lax.shift_right_logical(j, 1)
            return carry
        lax.fori_loop(0, BLK_IDX // 16, vec, 0)

    def start_gather(b):
        pltpu.async_copy(fmt_hbm.at[pidx[b]], rows[b], semg[b])

    def pack_block(k, b):
        iota2 = 2 * iota

        @plsc.parallel_loop(0, BLK_IDX, unroll=8)
        def row(g):
            jvec = plsc.load_gather(
                idx_v, [jnp.full((16,), k * BLK_IDX + g, jnp.int32)])
            par = jnp.bitwise_and(jvec, 1)
            grow = jnp.full((16,), g, jnp.int32)
            prow = jnp.full((16,), g // CTX, jnp.int32)
            cbase = (g % CTX) * EMB
            for q in range(4):
                vals = plsc.load_gather(rows[b], [grow, par + (iota2 + 2 * q * 16)])
                plsc.store_scatter(packs[b],
                                   [prow, iota + (cbase + q * 16)], vals)

    calc_pidx(0, 0)
    start_gather(0)

    def group(gp, carry):
        for b in range(2):
            k = gp * 2 + b

            @pl.when(k + 1 < BPW)
            def _():
                calc_pidx(k + 1, 1 - b)
                start_gather(1 - b)
            pltpu.make_async_copy(fmt_hbm.at[pidx[b]], rows[b],
                                  semg[b]).wait()
            pack_block(k, b)
            row0 = pl.multiple_of((wid * BPW + k) * 8, 8)
            pltpu.sync_copy(packs[b], out_hbm.at[pl.ds(row0, 8)])
        return carry

    lax.fori_loop(0, BPW // 2, group, 0)


def kernel(x, table):
    xp = x.astype(jnp.int32).reshape(-1)
    mesh = plsc.VectorSubcoreMesh(core_axis_name="c", subcore_axis_name="s")

    fmt = pl.kernel(
        _fmt_body,
        mesh=mesh,
        out_type=jax.ShapeDtypeStruct((FMT_ROWS, 128), jnp.float32),
        scratch_types=[
            pltpu.VMEM((EMB, CHUNK_V), jnp.float32),
            pltpu.VMEM((EMB, CHUNK_V), jnp.float32),
            pltpu.VMEM((ROWS_PC, 128), jnp.float32),
            pltpu.SemaphoreType.DMA,
            pltpu.SemaphoreType.DMA,
        ],
        compiler_params=pltpu.CompilerParams(use_tc_tiling_on_sc=True, needs_layout_passes=False),
    )(table.T, jnp.pad(table[FULL_COLS * 128:].T, ((0, 0), (0, 128 - TAIL_V))))

    out = pl.kernel(
        _gather_body,
        mesh=mesh,
        out_type=jax.ShapeDtypeStruct((B, CTX * EMB), jnp.float32),
        scratch_types=[
            pltpu.VMEM((PER_W,), jnp.int32),
            pltpu.VMEM((BLK_IDX,), jnp.int32),
            pltpu.VMEM((BLK_IDX,), jnp.int32),
            pltpu.VMEM((BLK_IDX, 128), jnp.float32),
            pltpu.VMEM((BLK_IDX, 128), jnp.float32),
            pltpu.VMEM((8, CTX * EMB), jnp.float32),
            pltpu.VMEM((8, CTX * EMB), jnp.float32),
            pltpu.SemaphoreType.DMA,
            pltpu.SemaphoreType.DMA,
        ],
        compiler_params=pltpu.CompilerParams(use_tc_tiling_on_sc=True, needs_layout_passes=False),
    )(xp, fmt)
    return out
